# baseline (speedup 1.0000x reference)
import jax
import jax.numpy as jnp
from jax import lax
from jax.experimental import pallas as pl
from jax.experimental.pallas import tpu as pltpu
from jax.experimental.pallas import tpu_sc as plsc

N_NODES = 10000
D = 128
E = 320000
NC = 2
NS = 16
NW = NC * NS
CH = 128
NCHUNK = E // CH
N_PAD = 10240
ROWS_PT = N_PAD // NS
DEGW = 128
N_PAIRS = 16384
P_TOT = 2 * N_PAIRS
PCH = 128
PNCH = P_TOT // (NW * PCH)


ACH = 64
K_RING = 4
ANCHUNK = E // ACH
A_FULL = (ANCHUNK // NW) // K_RING * K_RING
A_ITERS = A_FULL // K_RING
A_LEFT = ANCHUNK - A_FULL * NW


def _agg_body(h_hbm, src_hbm, dst_hbm, agg_out, *refs):
  sidx = refs[0:K_RING]
  didx = refs[K_RING:2 * K_RING]
  rows = refs[2 * K_RING:3 * K_RING]
  agg_sh = refs[3 * K_RING]
  gsem = refs[3 * K_RING + 1:3 * K_RING + 1 + K_RING]
  ssem = refs[3 * K_RING + 1 + K_RING:]
  c = lax.axis_index("c")
  s = lax.axis_index("s")
  t = s * NC + c
  zero16 = jnp.zeros((16,), jnp.float32)

  def zb(i, carry):
    rows[0][i // 8, pl.ds((i % 8) * 16, 16)] = zero16
    return carry
  lax.fori_loop(0, ACH * 8, zb, 0)

  def za(i, carry):
    pltpu.sync_copy(rows[0], agg_sh.at[pl.ds(s * ROWS_PT + i * ACH, ACH)])
    return carry
  lax.fori_loop(0, ROWS_PT // ACH, za, 0)

  plsc.subcore_barrier()

  for b in range(K_RING):
    base = (t + b * NW) * ACH
    pltpu.sync_copy(src_hbm.at[pl.ds(base, ACH)], sidx[b])
    pltpu.sync_copy(dst_hbm.at[pl.ds(base, ACH)], didx[b])
    pltpu.async_copy(h_hbm.at[sidx[b]], rows[b], gsem[b])

  def ring(g, carry):
    for b in range(K_RING):
      pltpu.make_async_copy(h_hbm.at[pl.ds(0, ACH)], rows[b], gsem[b]).wait()
      pltpu.async_copy(rows[b], agg_sh.at[didx[b]], ssem[b], add=True)
    for b in range(K_RING):
      pltpu.make_async_copy(rows[b], agg_sh.at[didx[b]], ssem[b]).wait()
      nxt = (g + 1) * K_RING + b

      @pl.when(nxt < A_FULL)
      def _prefetch():
        base = (t + nxt * NW) * ACH
        pltpu.sync_copy(src_hbm.at[pl.ds(base, ACH)], sidx[b])
        pltpu.sync_copy(dst_hbm.at[pl.ds(base, ACH)], didx[b])
        pltpu.async_copy(h_hbm.at[sidx[b]], rows[b], gsem[b])
    return carry
  lax.fori_loop(0, A_ITERS, ring, 0)

  @pl.when(t < A_LEFT)
  def _tail():
    base = (t + A_FULL * NW) * ACH
    pltpu.sync_copy(src_hbm.at[pl.ds(base, ACH)], sidx[0])
    pltpu.sync_copy(dst_hbm.at[pl.ds(base, ACH)], didx[0])
    pltpu.async_copy(h_hbm.at[sidx[0]], rows[0], gsem[0]).wait()
    pltpu.sync_copy(rows[0], agg_sh.at[didx[0]], add=True)

  plsc.subcore_barrier()

  def co(i, carry):
    r0 = s * ROWS_PT + i * ACH
    pltpu.sync_copy(agg_sh.at[pl.ds(r0, ACH)], rows[0])
    pltpu.sync_copy(rows[0], agg_out.at[pl.ds(c * N_PAD + r0, ACH)])
    return carry
  lax.fori_loop(0, ROWS_PT // ACH, co, 0)


_agg = pl.kernel(
    _agg_body,
    out_type=[jax.ShapeDtypeStruct((NC * N_PAD, D), jnp.float32)],
    mesh=plsc.VectorSubcoreMesh(core_axis_name="c", subcore_axis_name="s"),
    scratch_types=(
        [pltpu.VMEM((ACH,), jnp.int32) for _ in range(2 * K_RING)]
        + [pltpu.VMEM((ACH, D), jnp.float32) for _ in range(K_RING)]
        + [pltpu.VMEM_SHARED((N_PAD, D), jnp.float32)]
        + [pltpu.SemaphoreType.DMA for _ in range(2 * K_RING)]
    ),
)


def _deg_body(dst_hbm, deg_out, didx, ones, deg_sh):
  c = lax.axis_index("c")
  s = lax.axis_index("s")
  t = s * NC + c
  zero16 = jnp.zeros((16,), jnp.float32)

  def db(i, carry):
    ones[i // 8, pl.ds((i % 8) * 16, 16)] = zero16
    return carry
  lax.fori_loop(0, CH * 8, db, 0)

  def zd(i, carry):
    pltpu.sync_copy(ones, deg_sh.at[pl.ds(s * ROWS_PT + i * CH, CH)])
    return carry
  lax.fori_loop(0, ROWS_PT // CH, zd, 0)

  one16 = jnp.full((16,), 1.0, jnp.float32)

  def ob(i, carry):
    ones[i // 8, pl.ds((i % 8) * 16, 16)] = one16
    return carry
  lax.fori_loop(0, CH * 8, ob, 0)

  plsc.subcore_barrier()

  nch = NCHUNK // NW + jnp.where(t < NCHUNK % NW, 1, 0)

  def chunk(i, carry):
    base = (t + i * NW) * CH
    pltpu.sync_copy(dst_hbm.at[pl.ds(base, CH)], didx)
    pltpu.sync_copy(ones, deg_sh.at[didx], add=True)
    return carry
  lax.fori_loop(0, nch, chunk, 0)

  plsc.subcore_barrier()

  def cd(i, carry):
    r0 = s * ROWS_PT + i * CH
    pltpu.sync_copy(deg_sh.at[pl.ds(r0, CH)], ones)
    pltpu.sync_copy(ones, deg_out.at[pl.ds(c * N_PAD + r0, CH)])
    return carry
  lax.fori_loop(0, ROWS_PT // CH, cd, 0)


_deg = pl.kernel(
    _deg_body,
    out_type=[jax.ShapeDtypeStruct((NC * N_PAD, DEGW), jnp.float32)],
    mesh=plsc.VectorSubcoreMesh(core_axis_name="c", subcore_axis_name="s"),
    scratch_types=[
        pltpu.VMEM((CH,), jnp.int32),
        pltpu.VMEM((CH, DEGW), jnp.float32),
        pltpu.VMEM_SHARED((N_PAD, DEGW), jnp.float32),
    ],
)


_pairs_mesh = plsc.VectorSubcoreMesh(core_axis_name="c", subcore_axis_name="s")


def _pairs_body(h_hbm, a_hbm, b_hbm, out_hbm, aidx, bidx, ra, rb, sem, sem2):
  c = lax.axis_index("c")
  s = lax.axis_index("s")
  t = s * NC + c

  def chunk(i, carry):
    base = (t * PNCH + i) * PCH
    pltpu.sync_copy(a_hbm.at[pl.ds(base, PCH)], aidx)
    pltpu.sync_copy(b_hbm.at[pl.ds(base, PCH)], bidx)
    cp1 = pltpu.async_copy(h_hbm.at[aidx], ra, sem)
    cp2 = pltpu.async_copy(h_hbm.at[bidx], rb, sem2)
    cp1.wait()
    cp2.wait()

    def mul(j, carry2):
      r = j // 8
      o = (j % 8) * 16
      ra[r, pl.ds(o, 16)] = ra[r, pl.ds(o, 16)] * rb[r, pl.ds(o, 16)]
      return carry2
    lax.fori_loop(0, PCH * 8, mul, 0)
    pltpu.sync_copy(ra, out_hbm.at[pl.ds(base, PCH)])
    return carry
  lax.fori_loop(0, PNCH, chunk, 0)


_pairs = pl.kernel(
    _pairs_body,
    out_type=[jax.ShapeDtypeStruct((P_TOT, D), jnp.float32)],
    mesh=_pairs_mesh,
    scratch_types=[
        pltpu.VMEM((PCH,), jnp.int32),
        pltpu.VMEM((PCH,), jnp.int32),
        pltpu.VMEM((PCH, D), jnp.float32),
        pltpu.VMEM((PCH, D), jnp.float32),
        pltpu.SemaphoreType.DMA,
        pltpu.SemaphoreType.DMA,
    ],
)


def _layer_tc(h, parts, deg2, Ws, Wn, b, relu):
  n = h.shape[0]
  bm = 1000

  def body(h_ref, p_ref, d_ref, ws_ref, wn_ref, b_ref, o_ref):
    dcol = d_ref[0, :, 0:1] + d_ref[1, :, 0:1]
    hn = (p_ref[0] + p_ref[1]) / jnp.maximum(dcol, 1.0)
    acc = jnp.dot(h_ref[...], ws_ref[...], preferred_element_type=jnp.float32)
    acc = acc + jnp.dot(hn, wn_ref[...], preferred_element_type=jnp.float32)
    acc = acc + b_ref[...]
    if relu:
      acc = jnp.maximum(acc, 0.0)
    o_ref[...] = acc

  return pl.pallas_call(
      body,
      grid=(n // bm,),
      in_specs=[
          pl.BlockSpec((bm, D), lambda i: (i, 0)),
          pl.BlockSpec((NC, bm, D), lambda i: (0, i, 0)),
          pl.BlockSpec((NC, bm, DEGW), lambda i: (0, i, 0)),
          pl.BlockSpec((D, D), lambda i: (0, 0)),
          pl.BlockSpec((D, D), lambda i: (0, 0)),
          pl.BlockSpec((1, D), lambda i: (0, 0)),
      ],
      out_specs=pl.BlockSpec((bm, D), lambda i: (i, 0)),
      out_shape=jax.ShapeDtypeStruct((n, D), jnp.float32),
  )(h, parts, deg2, Ws, Wn, b)


def _pred_tc(prod, W1, c1, W2, c2, W3p, c3p):
  m = prod.shape[0]
  bm = 4096

  def body(x_ref, w1, b1, w2, b2, w3, b3, o_ref):
    h1 = jnp.dot(x_ref[...], w1[...], preferred_element_type=jnp.float32)
    h1 = jnp.maximum(h1 + b1[...], 0.0)
    h2 = jnp.dot(h1, w2[...], preferred_element_type=jnp.float32)
    h2 = jnp.maximum(h2 + b2[...], 0.0)
    o_ref[...] = jnp.dot(h2, w3[...], preferred_element_type=jnp.float32) + b3[...]

  return pl.pallas_call(
      body,
      grid=(m // bm,),
      in_specs=[
          pl.BlockSpec((bm, D), lambda i: (i, 0)),
          pl.BlockSpec((D, D), lambda i: (0, 0)),
          pl.BlockSpec((1, D), lambda i: (0, 0)),
          pl.BlockSpec((D, D), lambda i: (0, 0)),
          pl.BlockSpec((1, D), lambda i: (0, 0)),
          pl.BlockSpec((D, 8), lambda i: (0, 0)),
          pl.BlockSpec((1, 8), lambda i: (0, 0)),
      ],
      out_specs=pl.BlockSpec((bm, 8), lambda i: (i, 0)),
      out_shape=jax.ShapeDtypeStruct((m, 8), jnp.float32),
  )(prod, W1, c1, W2, c2, W3p, c3p)


def kernel(x, edge_index, pos_edge_index, neg_edge_index,
           W_self1, W_neigh1, b1, W_self2, W_neigh2, b2,
           W_self3, W_neigh3, b3,
           P1_W, P1_b, P2_W, P2_b, P3_W, P3_b):
  src = edge_index[0].astype(jnp.int32)
  dst = edge_index[1].astype(jnp.int32)

  parts1, = _agg(x, src, dst)
  parts1 = parts1.reshape(NC, N_PAD, D)
  deg2, = _deg(dst)
  deg2 = deg2.reshape(NC, N_PAD, DEGW)
  h1 = _layer_tc(x, parts1, deg2, W_self1, W_neigh1,
                 b1.reshape(1, D), relu=True)
  parts2, = _agg(h1, src, dst)
  parts2 = parts2.reshape(NC, N_PAD, D)
  h2 = _layer_tc(h1, parts2, deg2, W_self2, W_neigh2,
                 b2.reshape(1, D), relu=True)
  parts3, = _agg(h2, src, dst)
  parts3 = parts3.reshape(NC, N_PAD, D)
  h3 = _layer_tc(h2, parts3, deg2, W_self3, W_neigh3,
                 b3.reshape(1, D), relu=False)

  pair = jnp.concatenate([pos_edge_index, neg_edge_index], axis=1)
  a_idx = pair[0].astype(jnp.int32)
  b_idx = pair[1].astype(jnp.int32)
  prod, = _pairs(h3, a_idx, b_idx)

  W3p = jnp.pad(P3_W, ((0, 0), (0, 7)))
  c3p = jnp.pad(P3_b, (0, 7)).reshape(1, 8)
  out = _pred_tc(prod, P1_W, P1_b.reshape(1, D), P2_W, P2_b.reshape(1, D),
                 W3p, c3p)
  col = out[:, 0:1]
  return (col[:N_PAIRS], col[N_PAIRS:])


# speedup vs baseline: 1.1931x; 1.1931x over previous
import jax
import jax.numpy as jnp
from jax import lax
from jax.experimental import pallas as pl
from jax.experimental.pallas import tpu as pltpu
from jax.experimental.pallas import tpu_sc as plsc

N_NODES = 10000
D = 128
E = 320000
NC = 2
NS = 16
NW = NC * NS
CH = 128
NCHUNK = E // CH
N_PAD = 10240
ROWS_PT = N_PAD // NS
DEGW = 128
N_PAIRS = 16384
P_TOT = 2 * N_PAIRS
PCH = 128
PNCH = P_TOT // (NW * PCH)


G_FULL = (NCHUNK // NW) // 2 * 2
G_ITERS = G_FULL // 2
G_LEFT = NCHUNK - G_FULL * NW


def _agg_body(h_hbm, src_hbm, dst_hbm, agg_out,
              sa0, sa1, da0, da1, sb0, sb1, db0, db1, rows0, rows1, agg_sh,
              gsem0, gsem1, ssem0, ssem1, isem0, isem1):
  c = lax.axis_index("c")
  s = lax.axis_index("s")
  t = s * NC + c
  zero16 = jnp.zeros((16,), jnp.float32)
  set_a = ((sa0, sa1), (da0, da1))
  set_b = ((sb0, sb1), (db0, db1))
  rows = (rows0, rows1)
  gsem = (gsem0, gsem1)
  ssem = (ssem0, ssem1)
  isem = (isem0, isem1)

  def zb(i, carry):
    rows0[i // 8, pl.ds((i % 8) * 16, 16)] = zero16
    return carry
  lax.fori_loop(0, CH * 8, zb, 0)

  def za(i, carry):
    pltpu.sync_copy(rows0, agg_sh.at[pl.ds(s * ROWS_PT + i * CH, CH)])
    return carry
  lax.fori_loop(0, ROWS_PT // CH, za, 0)

  plsc.subcore_barrier()

  for b in range(2):
    base = (t + b * NW) * CH
    pltpu.sync_copy(src_hbm.at[pl.ds(base, CH)], set_a[0][b])
    pltpu.sync_copy(dst_hbm.at[pl.ds(base, CH)], set_a[1][b])
    pltpu.async_copy(h_hbm.at[set_a[0][b]], rows[b], gsem[b])

  def half(g, cur, nxt):
    cur_s, cur_d = cur
    nxt_s, nxt_d = nxt
    for b in range(2):
      pltpu.make_async_copy(h_hbm.at[pl.ds(0, CH)], rows[b], gsem[b]).wait()
      pltpu.async_copy(rows[b], agg_sh.at[cur_d[b]], ssem[b], add=True)
    for b in range(2):
      i_n = (g + 1) * 2 + b

      @pl.when(i_n < G_FULL)
      def _ld():
        base = (t + i_n * NW) * CH
        pltpu.async_copy(src_hbm.at[pl.ds(base, CH)], nxt_s[b], isem[b])
        pltpu.async_copy(dst_hbm.at[pl.ds(base, CH)], nxt_d[b], isem[b])
    for b in range(2):
      pltpu.make_async_copy(rows[b], agg_sh.at[cur_d[b]], ssem[b]).wait()
      i_n = (g + 1) * 2 + b

      @pl.when(i_n < G_FULL)
      def _gather():
        pltpu.make_async_copy(src_hbm.at[pl.ds(0, CH)], nxt_s[b], isem[b]).wait()
        pltpu.make_async_copy(dst_hbm.at[pl.ds(0, CH)], nxt_d[b], isem[b]).wait()
        pltpu.async_copy(h_hbm.at[nxt_s[b]], rows[b], gsem[b])

  def ring(g, carry):
    @pl.when(g % 2 == 0)
    def _even():
      half(g, set_a, set_b)

    @pl.when(g % 2 == 1)
    def _odd():
      half(g, set_b, set_a)
    return carry
  lax.fori_loop(0, G_ITERS, ring, 0)

  @pl.when(t < G_LEFT)
  def _tail():
    base = (t + G_FULL * NW) * CH
    pltpu.sync_copy(src_hbm.at[pl.ds(base, CH)], sa0)
    pltpu.sync_copy(dst_hbm.at[pl.ds(base, CH)], da0)
    pltpu.async_copy(h_hbm.at[sa0], rows0, gsem0).wait()
    pltpu.sync_copy(rows0, agg_sh.at[da0], add=True)

  plsc.subcore_barrier()

  def co(i, carry):
    r0 = s * ROWS_PT + i * CH
    pltpu.sync_copy(agg_sh.at[pl.ds(r0, CH)], rows0)
    pltpu.sync_copy(rows0, agg_out.at[pl.ds(c * N_PAD + r0, CH)])
    return carry
  lax.fori_loop(0, ROWS_PT // CH, co, 0)


_agg = pl.kernel(
    _agg_body,
    out_type=[jax.ShapeDtypeStruct((NC * N_PAD, D), jnp.float32)],
    mesh=plsc.VectorSubcoreMesh(core_axis_name="c", subcore_axis_name="s"),
    scratch_types=(
        [pltpu.VMEM((CH,), jnp.int32) for _ in range(8)]
        + [pltpu.VMEM((CH, D), jnp.float32) for _ in range(2)]
        + [pltpu.VMEM_SHARED((N_PAD, D), jnp.float32)]
        + [pltpu.SemaphoreType.DMA for _ in range(6)]
    ),
)


def _deg_body(dst_hbm, deg_out, didx, ones, deg_sh):
  c = lax.axis_index("c")
  s = lax.axis_index("s")
  t = s * NC + c
  zero16 = jnp.zeros((16,), jnp.float32)

  def db(i, carry):
    ones[i // 8, pl.ds((i % 8) * 16, 16)] = zero16
    return carry
  lax.fori_loop(0, CH * 8, db, 0)

  def zd(i, carry):
    pltpu.sync_copy(ones, deg_sh.at[pl.ds(s * ROWS_PT + i * CH, CH)])
    return carry
  lax.fori_loop(0, ROWS_PT // CH, zd, 0)

  one16 = jnp.full((16,), 1.0, jnp.float32)

  def ob(i, carry):
    ones[i // 8, pl.ds((i % 8) * 16, 16)] = one16
    return carry
  lax.fori_loop(0, CH * 8, ob, 0)

  plsc.subcore_barrier()

  nch = NCHUNK // NW + jnp.where(t < NCHUNK % NW, 1, 0)

  def chunk(i, carry):
    base = (t + i * NW) * CH
    pltpu.sync_copy(dst_hbm.at[pl.ds(base, CH)], didx)
    pltpu.sync_copy(ones, deg_sh.at[didx], add=True)
    return carry
  lax.fori_loop(0, nch, chunk, 0)

  plsc.subcore_barrier()

  def cd(i, carry):
    r0 = s * ROWS_PT + i * CH
    pltpu.sync_copy(deg_sh.at[pl.ds(r0, CH)], ones)
    pltpu.sync_copy(ones, deg_out.at[pl.ds(c * N_PAD + r0, CH)])
    return carry
  lax.fori_loop(0, ROWS_PT // CH, cd, 0)


_deg = pl.kernel(
    _deg_body,
    out_type=[jax.ShapeDtypeStruct((NC * N_PAD, DEGW), jnp.float32)],
    mesh=plsc.VectorSubcoreMesh(core_axis_name="c", subcore_axis_name="s"),
    scratch_types=[
        pltpu.VMEM((CH,), jnp.int32),
        pltpu.VMEM((CH, DEGW), jnp.float32),
        pltpu.VMEM_SHARED((N_PAD, DEGW), jnp.float32),
    ],
)


_pairs_mesh = plsc.VectorSubcoreMesh(core_axis_name="c", subcore_axis_name="s")


def _pairs_body(h_hbm, a_hbm, b_hbm, out_hbm, aidx, bidx, ra, rb, sem, sem2):
  c = lax.axis_index("c")
  s = lax.axis_index("s")
  t = s * NC + c

  def chunk(i, carry):
    base = (t * PNCH + i) * PCH
    pltpu.sync_copy(a_hbm.at[pl.ds(base, PCH)], aidx)
    pltpu.sync_copy(b_hbm.at[pl.ds(base, PCH)], bidx)
    cp1 = pltpu.async_copy(h_hbm.at[aidx], ra, sem)
    cp2 = pltpu.async_copy(h_hbm.at[bidx], rb, sem2)
    cp1.wait()
    cp2.wait()

    def mul(j, carry2):
      r = j // 8
      o = (j % 8) * 16
      ra[r, pl.ds(o, 16)] = ra[r, pl.ds(o, 16)] * rb[r, pl.ds(o, 16)]
      return carry2
    lax.fori_loop(0, PCH * 8, mul, 0)
    pltpu.sync_copy(ra, out_hbm.at[pl.ds(base, PCH)])
    return carry
  lax.fori_loop(0, PNCH, chunk, 0)


_pairs = pl.kernel(
    _pairs_body,
    out_type=[jax.ShapeDtypeStruct((P_TOT, D), jnp.float32)],
    mesh=_pairs_mesh,
    scratch_types=[
        pltpu.VMEM((PCH,), jnp.int32),
        pltpu.VMEM((PCH,), jnp.int32),
        pltpu.VMEM((PCH, D), jnp.float32),
        pltpu.VMEM((PCH, D), jnp.float32),
        pltpu.SemaphoreType.DMA,
        pltpu.SemaphoreType.DMA,
    ],
)


def _layer_tc(h, parts, deg2, Ws, Wn, b, relu):
  n = h.shape[0]
  bm = 1000

  def body(h_ref, p_ref, d_ref, ws_ref, wn_ref, b_ref, o_ref):
    dcol = d_ref[0, :, 0:1] + d_ref[1, :, 0:1]
    hn = (p_ref[0] + p_ref[1]) / jnp.maximum(dcol, 1.0)
    acc = jnp.dot(h_ref[...], ws_ref[...], preferred_element_type=jnp.float32)
    acc = acc + jnp.dot(hn, wn_ref[...], preferred_element_type=jnp.float32)
    acc = acc + b_ref[...]
    if relu:
      acc = jnp.maximum(acc, 0.0)
    o_ref[...] = acc

  return pl.pallas_call(
      body,
      grid=(n // bm,),
      in_specs=[
          pl.BlockSpec((bm, D), lambda i: (i, 0)),
          pl.BlockSpec((NC, bm, D), lambda i: (0, i, 0)),
          pl.BlockSpec((NC, bm, DEGW), lambda i: (0, i, 0)),
          pl.BlockSpec((D, D), lambda i: (0, 0)),
          pl.BlockSpec((D, D), lambda i: (0, 0)),
          pl.BlockSpec((1, D), lambda i: (0, 0)),
      ],
      out_specs=pl.BlockSpec((bm, D), lambda i: (i, 0)),
      out_shape=jax.ShapeDtypeStruct((n, D), jnp.float32),
  )(h, parts, deg2, Ws, Wn, b)


def _pred_tc(prod, W1, c1, W2, c2, W3p, c3p):
  m = prod.shape[0]
  bm = 4096

  def body(x_ref, w1, b1, w2, b2, w3, b3, o_ref):
    h1 = jnp.dot(x_ref[...], w1[...], preferred_element_type=jnp.float32)
    h1 = jnp.maximum(h1 + b1[...], 0.0)
    h2 = jnp.dot(h1, w2[...], preferred_element_type=jnp.float32)
    h2 = jnp.maximum(h2 + b2[...], 0.0)
    o_ref[...] = jnp.dot(h2, w3[...], preferred_element_type=jnp.float32) + b3[...]

  return pl.pallas_call(
      body,
      grid=(m // bm,),
      in_specs=[
          pl.BlockSpec((bm, D), lambda i: (i, 0)),
          pl.BlockSpec((D, D), lambda i: (0, 0)),
          pl.BlockSpec((1, D), lambda i: (0, 0)),
          pl.BlockSpec((D, D), lambda i: (0, 0)),
          pl.BlockSpec((1, D), lambda i: (0, 0)),
          pl.BlockSpec((D, 8), lambda i: (0, 0)),
          pl.BlockSpec((1, 8), lambda i: (0, 0)),
      ],
      out_specs=pl.BlockSpec((bm, 8), lambda i: (i, 0)),
      out_shape=jax.ShapeDtypeStruct((m, 8), jnp.float32),
  )(prod, W1, c1, W2, c2, W3p, c3p)


def kernel(x, edge_index, pos_edge_index, neg_edge_index,
           W_self1, W_neigh1, b1, W_self2, W_neigh2, b2,
           W_self3, W_neigh3, b3,
           P1_W, P1_b, P2_W, P2_b, P3_W, P3_b):
  src = edge_index[0].astype(jnp.int32)
  dst = edge_index[1].astype(jnp.int32)

  parts1, = _agg(x, src, dst)
  parts1 = parts1.reshape(NC, N_PAD, D)
  deg2, = _deg(dst)
  deg2 = deg2.reshape(NC, N_PAD, DEGW)
  h1 = _layer_tc(x, parts1, deg2, W_self1, W_neigh1,
                 b1.reshape(1, D), relu=True)
  parts2, = _agg(h1, src, dst)
  parts2 = parts2.reshape(NC, N_PAD, D)
  h2 = _layer_tc(h1, parts2, deg2, W_self2, W_neigh2,
                 b2.reshape(1, D), relu=True)
  parts3, = _agg(h2, src, dst)
  parts3 = parts3.reshape(NC, N_PAD, D)
  h3 = _layer_tc(h2, parts3, deg2, W_self3, W_neigh3,
                 b3.reshape(1, D), relu=False)

  pair = jnp.concatenate([pos_edge_index, neg_edge_index], axis=1)
  a_idx = pair[0].astype(jnp.int32)
  b_idx = pair[1].astype(jnp.int32)
  prod, = _pairs(h3, a_idx, b_idx)

  W3p = jnp.pad(P3_W, ((0, 0), (0, 7)))
  c3p = jnp.pad(P3_b, (0, 7)).reshape(1, 8)
  out = _pred_tc(prod, P1_W, P1_b.reshape(1, D), P2_W, P2_b.reshape(1, D),
                 W3p, c3p)
  col = out[:, 0:1]
  return (col[:N_PAIRS], col[N_PAIRS:])


# speedup vs baseline: 1.3760x; 1.1533x over previous
import jax
import jax.numpy as jnp
from jax import lax
from jax.experimental import pallas as pl
from jax.experimental.pallas import tpu as pltpu
from jax.experimental.pallas import tpu_sc as plsc

N_NODES = 10000
D = 128
E = 320000
NC = 2
NS = 16
NW = NC * NS
CH = 128
NCHUNK = E // CH
N_PAD = 10240
ROWS_PT = N_PAD // NS
DEGW = 128
N_PAIRS = 16384
P_TOT = 2 * N_PAIRS
PCH = 128
PNCH = P_TOT // (NW * PCH)


G_FULL = (NCHUNK // NW) // 2 * 2
G_ITERS = G_FULL // 2
G_LEFT = NCHUNK - G_FULL * NW


def _agg_body(h_hbm, src_hbm, dst_hbm, agg_out,
              sa0, sa1, da0, da1, sb0, sb1, db0, db1, rows0, rows1, agg_sh,
              gsem0, gsem1, ssem0, ssem1, isem0, isem1):
  c = lax.axis_index("c")
  s = lax.axis_index("s")
  t = s * NC + c
  zero16 = jnp.zeros((16,), jnp.float32)
  set_a = ((sa0, sa1), (da0, da1))
  set_b = ((sb0, sb1), (db0, db1))
  rows = (rows0, rows1)
  gsem = (gsem0, gsem1)
  ssem = (ssem0, ssem1)
  isem = (isem0, isem1)

  def zb(i, carry):
    rows0[i // 8, pl.ds((i % 8) * 16, 16)] = zero16
    return carry
  lax.fori_loop(0, CH * 8, zb, 0)

  def za(i, carry):
    pltpu.sync_copy(rows0, agg_sh.at[pl.ds(s * ROWS_PT + i * CH, CH)])
    return carry
  lax.fori_loop(0, ROWS_PT // CH, za, 0)

  plsc.subcore_barrier()

  for b in range(2):
    base = (t + b * NW) * CH
    pltpu.sync_copy(src_hbm.at[pl.ds(base, CH)], set_a[0][b])
    pltpu.sync_copy(dst_hbm.at[pl.ds(base, CH)], set_a[1][b])
    pltpu.async_copy(h_hbm.at[set_a[0][b]], rows[b], gsem[b])

  def half(g, cur, nxt):
    cur_s, cur_d = cur
    nxt_s, nxt_d = nxt
    for b in range(2):
      pltpu.make_async_copy(h_hbm.at[pl.ds(0, CH)], rows[b], gsem[b]).wait()
      pltpu.async_copy(rows[b], agg_sh.at[cur_d[b]], ssem[b], add=True)
    for b in range(2):
      i_n = (g + 1) * 2 + b

      @pl.when(i_n < G_FULL)
      def _ld():
        base = (t + i_n * NW) * CH
        pltpu.async_copy(src_hbm.at[pl.ds(base, CH)], nxt_s[b], isem[b])
        pltpu.async_copy(dst_hbm.at[pl.ds(base, CH)], nxt_d[b], isem[b])
    for b in range(2):
      pltpu.make_async_copy(rows[b], agg_sh.at[cur_d[b]], ssem[b]).wait()
      i_n = (g + 1) * 2 + b

      @pl.when(i_n < G_FULL)
      def _gather():
        pltpu.make_async_copy(src_hbm.at[pl.ds(0, CH)], nxt_s[b], isem[b]).wait()
        pltpu.make_async_copy(dst_hbm.at[pl.ds(0, CH)], nxt_d[b], isem[b]).wait()
        pltpu.async_copy(h_hbm.at[nxt_s[b]], rows[b], gsem[b])

  def ring(g, carry):
    @pl.when(g % 2 == 0)
    def _even():
      half(g, set_a, set_b)

    @pl.when(g % 2 == 1)
    def _odd():
      half(g, set_b, set_a)
    return carry
  lax.fori_loop(0, G_ITERS, ring, 0)

  @pl.when(t < G_LEFT)
  def _tail():
    base = (t + G_FULL * NW) * CH
    pltpu.sync_copy(src_hbm.at[pl.ds(base, CH)], sa0)
    pltpu.sync_copy(dst_hbm.at[pl.ds(base, CH)], da0)
    pltpu.async_copy(h_hbm.at[sa0], rows0, gsem0).wait()
    pltpu.sync_copy(rows0, agg_sh.at[da0], add=True)

  plsc.subcore_barrier()

  def co(i, carry):
    r0 = s * ROWS_PT + i * CH
    pltpu.sync_copy(agg_sh.at[pl.ds(r0, CH)], rows0)
    pltpu.sync_copy(rows0, agg_out.at[pl.ds(c * N_PAD + r0, CH)])
    return carry
  lax.fori_loop(0, ROWS_PT // CH, co, 0)


_agg = pl.kernel(
    _agg_body,
    out_type=[jax.ShapeDtypeStruct((NC * N_PAD, D), jnp.float32)],
    mesh=plsc.VectorSubcoreMesh(core_axis_name="c", subcore_axis_name="s"),
    scratch_types=(
        [pltpu.VMEM((CH,), jnp.int32) for _ in range(8)]
        + [pltpu.VMEM((CH, D), jnp.float32) for _ in range(2)]
        + [pltpu.VMEM_SHARED((N_PAD, D), jnp.float32)]
        + [pltpu.SemaphoreType.DMA for _ in range(6)]
    ),
)


def _deg_body(dst_hbm, deg_out, didx, ones1, tmp1, deg_sh):
  c = lax.axis_index("c")
  s = lax.axis_index("s")
  t = s * NC + c
  zero16 = jnp.zeros((16,), jnp.float32)

  def db(i, carry):
    ones1[pl.ds(i * 16, 16)] = zero16
    return carry
  lax.fori_loop(0, CH // 16, db, 0)

  def zd(i, carry):
    pltpu.sync_copy(ones1, deg_sh.at[pl.ds(s * ROWS_PT + i * CH, CH)])
    return carry
  lax.fori_loop(0, ROWS_PT // CH, zd, 0)

  one16 = jnp.full((16,), 1.0, jnp.float32)

  def ob(i, carry):
    ones1[pl.ds(i * 16, 16)] = one16
    return carry
  lax.fori_loop(0, CH // 16, ob, 0)

  plsc.subcore_barrier()

  nch = NCHUNK // NW + jnp.where(t < NCHUNK % NW, 1, 0)

  def chunk(i, carry):
    base = (t + i * NW) * CH
    pltpu.sync_copy(dst_hbm.at[pl.ds(base, CH)], didx)
    pltpu.sync_copy(ones1, deg_sh.at[didx], add=True)
    return carry
  lax.fori_loop(0, nch, chunk, 0)

  plsc.subcore_barrier()

  def cd(i, carry):
    r0 = s * ROWS_PT + i * CH
    pltpu.sync_copy(deg_sh.at[pl.ds(r0, CH)], tmp1)
    pltpu.sync_copy(tmp1, deg_out.at[pl.ds(c * N_PAD + r0, CH)])
    return carry
  lax.fori_loop(0, ROWS_PT // CH, cd, 0)


_deg = pl.kernel(
    _deg_body,
    out_type=[jax.ShapeDtypeStruct((NC * N_PAD,), jnp.float32)],
    mesh=plsc.VectorSubcoreMesh(core_axis_name="c", subcore_axis_name="s"),
    scratch_types=[
        pltpu.VMEM((CH,), jnp.int32),
        pltpu.VMEM((CH,), jnp.float32),
        pltpu.VMEM((CH,), jnp.float32),
        pltpu.VMEM_SHARED((N_PAD,), jnp.float32),
    ],
)


_pairs_mesh = plsc.VectorSubcoreMesh(core_axis_name="c", subcore_axis_name="s")


def _pairs_body(h_hbm, a_hbm, b_hbm, a_out, b_out,
                aidx0, aidx1, bidx0, bidx1, ra0, ra1, rb0, rb1,
                gsem0, gsem1, osem0, osem1):
  c = lax.axis_index("c")
  s = lax.axis_index("s")
  t = s * NC + c
  aidx = (aidx0, aidx1)
  bidx = (bidx0, bidx1)
  ra = (ra0, ra1)
  rb = (rb0, rb1)
  gsem = (gsem0, gsem1)
  osem = (osem0, osem1)

  def chunk_pair(g, carry):
    for b in range(2):
      i = g * 2 + b
      base = (t * PNCH + i) * PCH

      @pl.when(g > 0)
      def _drain():
        pltpu.make_async_copy(ra[b], a_out.at[pl.ds(0, PCH)], osem[b]).wait()
        pltpu.make_async_copy(rb[b], b_out.at[pl.ds(0, PCH)], osem[b]).wait()

      pltpu.sync_copy(a_hbm.at[pl.ds(base, PCH)], aidx[b])
      pltpu.sync_copy(b_hbm.at[pl.ds(base, PCH)], bidx[b])
      pltpu.async_copy(h_hbm.at[aidx[b]], ra[b], gsem[b])
      pltpu.async_copy(h_hbm.at[bidx[b]], rb[b], gsem[b])
    for b in range(2):
      i = g * 2 + b
      base = (t * PNCH + i) * PCH
      pltpu.make_async_copy(h_hbm.at[pl.ds(0, PCH)], ra[b], gsem[b]).wait()
      pltpu.make_async_copy(h_hbm.at[pl.ds(0, PCH)], rb[b], gsem[b]).wait()
      pltpu.async_copy(ra[b], a_out.at[pl.ds(base, PCH)], osem[b])
      pltpu.async_copy(rb[b], b_out.at[pl.ds(base, PCH)], osem[b])
    return carry
  lax.fori_loop(0, PNCH // 2, chunk_pair, 0)

  for b in range(2):
    pltpu.make_async_copy(ra[b], a_out.at[pl.ds(0, PCH)], osem[b]).wait()
    pltpu.make_async_copy(rb[b], b_out.at[pl.ds(0, PCH)], osem[b]).wait()


_pairs = pl.kernel(
    _pairs_body,
    out_type=[jax.ShapeDtypeStruct((P_TOT, D), jnp.float32),
              jax.ShapeDtypeStruct((P_TOT, D), jnp.float32)],
    mesh=_pairs_mesh,
    scratch_types=(
        [pltpu.VMEM((PCH,), jnp.int32) for _ in range(4)]
        + [pltpu.VMEM((PCH, D), jnp.float32) for _ in range(4)]
        + [pltpu.SemaphoreType.DMA for _ in range(4)]
    ),
)


def _layer_tc(h, parts, deg2, Ws, Wn, b, relu):
  n = h.shape[0]
  bm = 1000

  def body(h_ref, p_ref, d_ref, ws_ref, wn_ref, b_ref, o_ref):
    dcol = d_ref[0] + d_ref[1]
    hn = (p_ref[0] + p_ref[1]) / jnp.maximum(dcol, 1.0)
    acc = jnp.dot(h_ref[...], ws_ref[...], preferred_element_type=jnp.float32)
    acc = acc + jnp.dot(hn, wn_ref[...], preferred_element_type=jnp.float32)
    acc = acc + b_ref[...]
    if relu:
      acc = jnp.maximum(acc, 0.0)
    o_ref[...] = acc

  return pl.pallas_call(
      body,
      grid=(n // bm,),
      in_specs=[
          pl.BlockSpec((bm, D), lambda i: (i, 0)),
          pl.BlockSpec((NC, bm, D), lambda i: (0, i, 0)),
          pl.BlockSpec((NC, bm, 1), lambda i: (0, i, 0)),
          pl.BlockSpec((D, D), lambda i: (0, 0)),
          pl.BlockSpec((D, D), lambda i: (0, 0)),
          pl.BlockSpec((1, D), lambda i: (0, 0)),
      ],
      out_specs=pl.BlockSpec((bm, D), lambda i: (i, 0)),
      out_shape=jax.ShapeDtypeStruct((n, D), jnp.float32),
  )(h, parts, deg2, Ws, Wn, b)


def _pred_tc(arows, brows, W1, c1, W2, c2, W3p, c3p):
  m = arows.shape[0]
  bm = 4096

  def body(a_ref, b_ref, w1, b1, w2, b2, w3, b3, o_ref):
    x = a_ref[...] * b_ref[...]
    h1 = jnp.dot(x, w1[...], preferred_element_type=jnp.float32)
    h1 = jnp.maximum(h1 + b1[...], 0.0)
    h2 = jnp.dot(h1, w2[...], preferred_element_type=jnp.float32)
    h2 = jnp.maximum(h2 + b2[...], 0.0)
    o_ref[...] = jnp.dot(h2, w3[...], preferred_element_type=jnp.float32) + b3[...]

  return pl.pallas_call(
      body,
      grid=(m // bm,),
      in_specs=[
          pl.BlockSpec((bm, D), lambda i: (i, 0)),
          pl.BlockSpec((bm, D), lambda i: (i, 0)),
          pl.BlockSpec((D, D), lambda i: (0, 0)),
          pl.BlockSpec((1, D), lambda i: (0, 0)),
          pl.BlockSpec((D, D), lambda i: (0, 0)),
          pl.BlockSpec((1, D), lambda i: (0, 0)),
          pl.BlockSpec((D, 8), lambda i: (0, 0)),
          pl.BlockSpec((1, 8), lambda i: (0, 0)),
      ],
      out_specs=pl.BlockSpec((bm, 8), lambda i: (i, 0)),
      out_shape=jax.ShapeDtypeStruct((m, 8), jnp.float32),
  )(arows, brows, W1, c1, W2, c2, W3p, c3p)


def kernel(x, edge_index, pos_edge_index, neg_edge_index,
           W_self1, W_neigh1, b1, W_self2, W_neigh2, b2,
           W_self3, W_neigh3, b3,
           P1_W, P1_b, P2_W, P2_b, P3_W, P3_b):
  src = edge_index[0].astype(jnp.int32)
  dst = edge_index[1].astype(jnp.int32)

  parts1, = _agg(x, src, dst)
  parts1 = parts1.reshape(NC, N_PAD, D)
  deg2, = _deg(dst)
  deg2 = deg2.reshape(NC, N_PAD, 1)
  h1 = _layer_tc(x, parts1, deg2, W_self1, W_neigh1,
                 b1.reshape(1, D), relu=True)
  parts2, = _agg(h1, src, dst)
  parts2 = parts2.reshape(NC, N_PAD, D)
  h2 = _layer_tc(h1, parts2, deg2, W_self2, W_neigh2,
                 b2.reshape(1, D), relu=True)
  parts3, = _agg(h2, src, dst)
  parts3 = parts3.reshape(NC, N_PAD, D)
  h3 = _layer_tc(h2, parts3, deg2, W_self3, W_neigh3,
                 b3.reshape(1, D), relu=False)

  pair = jnp.concatenate([pos_edge_index, neg_edge_index], axis=1)
  a_idx = pair[0].astype(jnp.int32)
  b_idx = pair[1].astype(jnp.int32)
  arows, brows = _pairs(h3, a_idx, b_idx)

  W3p = jnp.pad(P3_W, ((0, 0), (0, 7)))
  c3p = jnp.pad(P3_b, (0, 7)).reshape(1, 8)
  out = _pred_tc(arows, brows, P1_W, P1_b.reshape(1, D),
                 P2_W, P2_b.reshape(1, D), W3p, c3p)
  col = out[:, 0:1]
  return (col[:N_PAIRS], col[N_PAIRS:])


# speedup vs baseline: 1.4773x; 1.0736x over previous
import jax
import jax.numpy as jnp
from jax import lax
from jax.experimental import pallas as pl
from jax.experimental.pallas import tpu as pltpu
from jax.experimental.pallas import tpu_sc as plsc

N_NODES = 10000
D = 128
E = 320000
NC = 2
NS = 16
NW = NC * NS
CH = 128
NCHUNK = E // CH
N_PAD = 10240
ROWS_PT = N_PAD // NS
DEGW = 128
N_PAIRS = 16384
P_TOT = 2 * N_PAIRS
PCH = 128
PNCH = P_TOT // (NW * PCH)


G_FULL = (NCHUNK // NW) // 2 * 2
G_ITERS = G_FULL // 2
G_LEFT = NCHUNK - G_FULL * NW


def _make_agg_body(with_deg):
  def body(h_hbm, src_hbm, dst_hbm, agg_out, *rest):
    if with_deg:
      (deg_out, sa0, sa1, da0, da1, sb0, sb1, db0, db1, rows0, rows1,
       ones1, agg_sh, deg_sh, gsem0, gsem1, ssem0, ssem1, isem0, isem1,
       dsem0, dsem1) = rest
      dsem = (dsem0, dsem1)
    else:
      (sa0, sa1, da0, da1, sb0, sb1, db0, db1, rows0, rows1, agg_sh,
       gsem0, gsem1, ssem0, ssem1, isem0, isem1) = rest
    c = lax.axis_index("c")
    s = lax.axis_index("s")
    t = s * NC + c
    zero16 = jnp.zeros((16,), jnp.float32)
    set_a = ((sa0, sa1), (da0, da1))
    set_b = ((sb0, sb1), (db0, db1))
    rows = (rows0, rows1)
    gsem = (gsem0, gsem1)
    ssem = (ssem0, ssem1)
    isem = (isem0, isem1)

    def zb(i, carry):
      rows0[i // 8, pl.ds((i % 8) * 16, 16)] = zero16
      return carry
    lax.fori_loop(0, CH * 8, zb, 0)

    def za(i, carry):
      pltpu.sync_copy(rows0, agg_sh.at[pl.ds(s * ROWS_PT + i * CH, CH)])
      return carry
    lax.fori_loop(0, ROWS_PT // CH, za, 0)

    if with_deg:
      def zo(i, carry):
        ones1[pl.ds(i * 16, 16)] = zero16
        return carry
      lax.fori_loop(0, CH // 16, zo, 0)

      def zd(i, carry):
        pltpu.sync_copy(ones1, deg_sh.at[pl.ds(s * ROWS_PT + i * CH, CH)])
        return carry
      lax.fori_loop(0, ROWS_PT // CH, zd, 0)

      one16 = jnp.full((16,), 1.0, jnp.float32)

      def fo(i, carry):
        ones1[pl.ds(i * 16, 16)] = one16
        return carry
      lax.fori_loop(0, CH // 16, fo, 0)

    plsc.subcore_barrier()

    for b in range(2):
      base = (t + b * NW) * CH
      pltpu.sync_copy(src_hbm.at[pl.ds(base, CH)], set_a[0][b])
      pltpu.sync_copy(dst_hbm.at[pl.ds(base, CH)], set_a[1][b])
      pltpu.async_copy(h_hbm.at[set_a[0][b]], rows[b], gsem[b])

    def half(g, cur, nxt):
      cur_s, cur_d = cur
      nxt_s, nxt_d = nxt
      for b in range(2):
        pltpu.make_async_copy(h_hbm.at[pl.ds(0, CH)], rows[b], gsem[b]).wait()
        pltpu.async_copy(rows[b], agg_sh.at[cur_d[b]], ssem[b], add=True)
        if with_deg:
          pltpu.async_copy(ones1, deg_sh.at[cur_d[b]], dsem[b], add=True)
      for b in range(2):
        i_n = (g + 1) * 2 + b

        @pl.when(i_n < G_FULL)
        def _ld():
          base = (t + i_n * NW) * CH
          pltpu.async_copy(src_hbm.at[pl.ds(base, CH)], nxt_s[b], isem[b])
          pltpu.async_copy(dst_hbm.at[pl.ds(base, CH)], nxt_d[b], isem[b])
      for b in range(2):
        pltpu.make_async_copy(rows[b], agg_sh.at[cur_d[b]], ssem[b]).wait()
        if with_deg:
          pltpu.make_async_copy(ones1, deg_sh.at[cur_d[b]], dsem[b]).wait()
        i_n = (g + 1) * 2 + b

        @pl.when(i_n < G_FULL)
        def _gather():
          pltpu.make_async_copy(src_hbm.at[pl.ds(0, CH)], nxt_s[b], isem[b]).wait()
          pltpu.make_async_copy(dst_hbm.at[pl.ds(0, CH)], nxt_d[b], isem[b]).wait()
          pltpu.async_copy(h_hbm.at[nxt_s[b]], rows[b], gsem[b])

    def ring(g, carry):
      @pl.when(g % 2 == 0)
      def _even():
        half(g, set_a, set_b)

      @pl.when(g % 2 == 1)
      def _odd():
        half(g, set_b, set_a)
      return carry
    lax.fori_loop(0, G_ITERS, ring, 0)

    @pl.when(t < G_LEFT)
    def _tail():
      base = (t + G_FULL * NW) * CH
      pltpu.sync_copy(src_hbm.at[pl.ds(base, CH)], sa0)
      pltpu.sync_copy(dst_hbm.at[pl.ds(base, CH)], da0)
      pltpu.async_copy(h_hbm.at[sa0], rows0, gsem0).wait()
      pltpu.sync_copy(rows0, agg_sh.at[da0], add=True)
      if with_deg:
        pltpu.sync_copy(ones1, deg_sh.at[da0], add=True)

    plsc.subcore_barrier()

    def co(i, carry):
      r0 = s * ROWS_PT + i * CH
      pltpu.sync_copy(agg_sh.at[pl.ds(r0, CH)], rows0)
      pltpu.sync_copy(rows0, agg_out.at[pl.ds(c * N_PAD + r0, CH)])
      return carry
    lax.fori_loop(0, ROWS_PT // CH, co, 0)

    if with_deg:
      def cd(i, carry):
        r0 = s * ROWS_PT + i * CH
        pltpu.sync_copy(deg_sh.at[pl.ds(r0, CH)], ones1)
        pltpu.sync_copy(ones1, deg_out.at[pl.ds(c * N_PAD + r0, CH)])
        return carry
      lax.fori_loop(0, ROWS_PT // CH, cd, 0)

  return body


def _make_agg(with_deg):
  out_type = [jax.ShapeDtypeStruct((NC * N_PAD, D), jnp.float32)]
  scratch = (
      [pltpu.VMEM((CH,), jnp.int32) for _ in range(8)]
      + [pltpu.VMEM((CH, D), jnp.float32) for _ in range(2)]
  )
  if with_deg:
    out_type.append(jax.ShapeDtypeStruct((NC * N_PAD,), jnp.float32))
    scratch = scratch + [pltpu.VMEM((CH,), jnp.float32)]
  scratch = scratch + [pltpu.VMEM_SHARED((N_PAD, D), jnp.float32)]
  if with_deg:
    scratch = scratch + [pltpu.VMEM_SHARED((N_PAD,), jnp.float32)]
  scratch = scratch + [pltpu.SemaphoreType.DMA
                       for _ in range(8 if with_deg else 6)]
  return pl.kernel(
      _make_agg_body(with_deg),
      out_type=out_type,
      mesh=plsc.VectorSubcoreMesh(core_axis_name="c", subcore_axis_name="s"),
      scratch_types=scratch,
  )


_agg = _make_agg(False)
_agg_deg = _make_agg(True)


_pairs_mesh = plsc.VectorSubcoreMesh(core_axis_name="c", subcore_axis_name="s")


def _pairs_body(h_hbm, a_hbm, b_hbm, a_out, b_out,
                aidx0, aidx1, bidx0, bidx1, ra0, ra1, rb0, rb1,
                gsem0, gsem1, osem0, osem1):
  c = lax.axis_index("c")
  s = lax.axis_index("s")
  t = s * NC + c
  aidx = (aidx0, aidx1)
  bidx = (bidx0, bidx1)
  ra = (ra0, ra1)
  rb = (rb0, rb1)
  gsem = (gsem0, gsem1)
  osem = (osem0, osem1)

  def chunk_pair(g, carry):
    for b in range(2):
      i = g * 2 + b
      base = (t * PNCH + i) * PCH

      @pl.when(g > 0)
      def _drain():
        pltpu.make_async_copy(ra[b], a_out.at[pl.ds(0, PCH)], osem[b]).wait()
        pltpu.make_async_copy(rb[b], b_out.at[pl.ds(0, PCH)], osem[b]).wait()

      pltpu.sync_copy(a_hbm.at[pl.ds(base, PCH)], aidx[b])
      pltpu.sync_copy(b_hbm.at[pl.ds(base, PCH)], bidx[b])
      pltpu.async_copy(h_hbm.at[aidx[b]], ra[b], gsem[b])
      pltpu.async_copy(h_hbm.at[bidx[b]], rb[b], gsem[b])
    for b in range(2):
      i = g * 2 + b
      base = (t * PNCH + i) * PCH
      pltpu.make_async_copy(h_hbm.at[pl.ds(0, PCH)], ra[b], gsem[b]).wait()
      pltpu.make_async_copy(h_hbm.at[pl.ds(0, PCH)], rb[b], gsem[b]).wait()
      pltpu.async_copy(ra[b], a_out.at[pl.ds(base, PCH)], osem[b])
      pltpu.async_copy(rb[b], b_out.at[pl.ds(base, PCH)], osem[b])
    return carry
  lax.fori_loop(0, PNCH // 2, chunk_pair, 0)

  for b in range(2):
    pltpu.make_async_copy(ra[b], a_out.at[pl.ds(0, PCH)], osem[b]).wait()
    pltpu.make_async_copy(rb[b], b_out.at[pl.ds(0, PCH)], osem[b]).wait()


_pairs = pl.kernel(
    _pairs_body,
    out_type=[jax.ShapeDtypeStruct((P_TOT, D), jnp.float32),
              jax.ShapeDtypeStruct((P_TOT, D), jnp.float32)],
    mesh=_pairs_mesh,
    scratch_types=(
        [pltpu.VMEM((PCH,), jnp.int32) for _ in range(4)]
        + [pltpu.VMEM((PCH, D), jnp.float32) for _ in range(4)]
        + [pltpu.SemaphoreType.DMA for _ in range(4)]
    ),
)


def _layer_tc(h, parts, deg2, Ws, Wn, b, relu):
  n = h.shape[0]
  bm = 1000

  def body(h_ref, p_ref, d_ref, ws_ref, wn_ref, b_ref, o_ref):
    dcol = d_ref[0] + d_ref[1]
    hn = (p_ref[0] + p_ref[1]) / jnp.maximum(dcol, 1.0)
    acc = jnp.dot(h_ref[...], ws_ref[...], preferred_element_type=jnp.float32)
    acc = acc + jnp.dot(hn, wn_ref[...], preferred_element_type=jnp.float32)
    acc = acc + b_ref[...]
    if relu:
      acc = jnp.maximum(acc, 0.0)
    o_ref[...] = acc

  return pl.pallas_call(
      body,
      grid=(n // bm,),
      in_specs=[
          pl.BlockSpec((bm, D), lambda i: (i, 0)),
          pl.BlockSpec((NC, bm, D), lambda i: (0, i, 0)),
          pl.BlockSpec((NC, bm, 1), lambda i: (0, i, 0)),
          pl.BlockSpec((D, D), lambda i: (0, 0)),
          pl.BlockSpec((D, D), lambda i: (0, 0)),
          pl.BlockSpec((1, D), lambda i: (0, 0)),
      ],
      out_specs=pl.BlockSpec((bm, D), lambda i: (i, 0)),
      out_shape=jax.ShapeDtypeStruct((n, D), jnp.float32),
  )(h, parts, deg2, Ws, Wn, b)


def _pred_tc(arows, brows, W1, c1, W2, c2, W3p, c3p):
  m = arows.shape[0]
  bm = 4096

  def body(a_ref, b_ref, w1, b1, w2, b2, w3, b3, o_ref):
    x = a_ref[...] * b_ref[...]
    h1 = jnp.dot(x, w1[...], preferred_element_type=jnp.float32)
    h1 = jnp.maximum(h1 + b1[...], 0.0)
    h2 = jnp.dot(h1, w2[...], preferred_element_type=jnp.float32)
    h2 = jnp.maximum(h2 + b2[...], 0.0)
    o_ref[...] = jnp.dot(h2, w3[...], preferred_element_type=jnp.float32) + b3[...]

  return pl.pallas_call(
      body,
      grid=(m // bm,),
      in_specs=[
          pl.BlockSpec((bm, D), lambda i: (i, 0)),
          pl.BlockSpec((bm, D), lambda i: (i, 0)),
          pl.BlockSpec((D, D), lambda i: (0, 0)),
          pl.BlockSpec((1, D), lambda i: (0, 0)),
          pl.BlockSpec((D, D), lambda i: (0, 0)),
          pl.BlockSpec((1, D), lambda i: (0, 0)),
          pl.BlockSpec((D, 8), lambda i: (0, 0)),
          pl.BlockSpec((1, 8), lambda i: (0, 0)),
      ],
      out_specs=pl.BlockSpec((bm, 8), lambda i: (i, 0)),
      out_shape=jax.ShapeDtypeStruct((m, 8), jnp.float32),
  )(arows, brows, W1, c1, W2, c2, W3p, c3p)


def kernel(x, edge_index, pos_edge_index, neg_edge_index,
           W_self1, W_neigh1, b1, W_self2, W_neigh2, b2,
           W_self3, W_neigh3, b3,
           P1_W, P1_b, P2_W, P2_b, P3_W, P3_b):
  src = edge_index[0].astype(jnp.int32)
  dst = edge_index[1].astype(jnp.int32)

  parts1, deg1d = _agg_deg(x, src, dst)
  parts1 = parts1.reshape(NC, N_PAD, D)
  deg2 = deg1d.reshape(NC, N_PAD, 1)
  h1 = _layer_tc(x, parts1, deg2, W_self1, W_neigh1,
                 b1.reshape(1, D), relu=True)
  parts2, = _agg(h1, src, dst)
  parts2 = parts2.reshape(NC, N_PAD, D)
  h2 = _layer_tc(h1, parts2, deg2, W_self2, W_neigh2,
                 b2.reshape(1, D), relu=True)
  parts3, = _agg(h2, src, dst)
  parts3 = parts3.reshape(NC, N_PAD, D)
  h3 = _layer_tc(h2, parts3, deg2, W_self3, W_neigh3,
                 b3.reshape(1, D), relu=False)

  pair = jnp.concatenate([pos_edge_index, neg_edge_index], axis=1)
  a_idx = pair[0].astype(jnp.int32)
  b_idx = pair[1].astype(jnp.int32)
  arows, brows = _pairs(h3, a_idx, b_idx)

  W3p = jnp.pad(P3_W, ((0, 0), (0, 7)))
  c3p = jnp.pad(P3_b, (0, 7)).reshape(1, 8)
  out = _pred_tc(arows, brows, P1_W, P1_b.reshape(1, D),
                 P2_W, P2_b.reshape(1, D), W3p, c3p)
  col = out[:, 0:1]
  return (col[:N_PAIRS], col[N_PAIRS:])


# speedup vs baseline: 1.4979x; 1.0139x over previous
import jax
import jax.numpy as jnp
from jax import lax
from jax.experimental import pallas as pl
from jax.experimental.pallas import tpu as pltpu
from jax.experimental.pallas import tpu_sc as plsc

N_NODES = 10000
D = 128
E = 320000
NC = 2
NS = 16
NW = NC * NS
CH = 128
NCHUNK = E // CH
N_PAD = 10240
ROWS_PT = N_PAD // NS
DEGW = 128
N_PAIRS = 16384
P_TOT = 2 * N_PAIRS
PCH = 128
PNCH = P_TOT // (NW * PCH)


G_FULL = (NCHUNK // NW) // 2 * 2
G_ITERS = G_FULL // 2
G_LEFT = NCHUNK - G_FULL * NW


def _make_agg_body(with_deg):
  def body(h_hbm, src_hbm, dst_hbm, agg_out, *rest):
    if with_deg:
      (deg_out, sa0, sa1, da0, da1, sb0, sb1, db0, db1, rows0, rows1,
       ones1, agg_sh, deg_sh, gsem0, gsem1, ssem0, ssem1, isem0, isem1,
       dsem0, dsem1) = rest
      dsem = (dsem0, dsem1)
    else:
      (sa0, sa1, da0, da1, sb0, sb1, db0, db1, rows0, rows1, agg_sh,
       gsem0, gsem1, ssem0, ssem1, isem0, isem1) = rest
    c = lax.axis_index("c")
    s = lax.axis_index("s")
    t = s * NC + c
    zero16 = jnp.zeros((16,), jnp.float32)
    set_a = ((sa0, sa1), (da0, da1))
    set_b = ((sb0, sb1), (db0, db1))
    rows = (rows0, rows1)
    gsem = (gsem0, gsem1)
    ssem = (ssem0, ssem1)
    isem = (isem0, isem1)

    def zb(i, carry):
      rows0[i // 8, pl.ds((i % 8) * 16, 16)] = zero16
      return carry
    lax.fori_loop(0, CH * 8, zb, 0)

    zcps = []
    for i in range(ROWS_PT // CH):
      zcps.append(pltpu.async_copy(
          rows0, agg_sh.at[pl.ds(s * ROWS_PT + i * CH, CH)], gsem0))
    if with_deg:
      def zo(i, carry):
        ones1[pl.ds(i * 16, 16)] = zero16
        return carry
      lax.fori_loop(0, CH // 16, zo, 0)

      for i in range(ROWS_PT // CH):
        zcps.append(pltpu.async_copy(
            ones1, deg_sh.at[pl.ds(s * ROWS_PT + i * CH, CH)], gsem1))
    for cp in zcps:
      cp.wait()
    if with_deg:
      one16 = jnp.full((16,), 1.0, jnp.float32)

      def fo(i, carry):
        ones1[pl.ds(i * 16, 16)] = one16
        return carry
      lax.fori_loop(0, CH // 16, fo, 0)

    plsc.subcore_barrier()

    for b in range(2):
      base = (t + b * NW) * CH
      pltpu.sync_copy(src_hbm.at[pl.ds(base, CH)], set_a[0][b])
      pltpu.sync_copy(dst_hbm.at[pl.ds(base, CH)], set_a[1][b])
      pltpu.async_copy(h_hbm.at[set_a[0][b]], rows[b], gsem[b])

    def half(g, cur, nxt):
      cur_s, cur_d = cur
      nxt_s, nxt_d = nxt
      for b in range(2):
        pltpu.make_async_copy(h_hbm.at[pl.ds(0, CH)], rows[b], gsem[b]).wait()
        pltpu.async_copy(rows[b], agg_sh.at[cur_d[b]], ssem[b], add=True)
        if with_deg:
          pltpu.async_copy(ones1, deg_sh.at[cur_d[b]], dsem[b], add=True)
      for b in range(2):
        i_n = (g + 1) * 2 + b

        @pl.when(i_n < G_FULL)
        def _ld():
          base = (t + i_n * NW) * CH
          pltpu.async_copy(src_hbm.at[pl.ds(base, CH)], nxt_s[b], isem[b])
          pltpu.async_copy(dst_hbm.at[pl.ds(base, CH)], nxt_d[b], isem[b])
      for b in range(2):
        pltpu.make_async_copy(rows[b], agg_sh.at[cur_d[b]], ssem[b]).wait()
        if with_deg:
          pltpu.make_async_copy(ones1, deg_sh.at[cur_d[b]], dsem[b]).wait()
        i_n = (g + 1) * 2 + b

        @pl.when(i_n < G_FULL)
        def _gather():
          pltpu.make_async_copy(src_hbm.at[pl.ds(0, CH)], nxt_s[b], isem[b]).wait()
          pltpu.make_async_copy(dst_hbm.at[pl.ds(0, CH)], nxt_d[b], isem[b]).wait()
          pltpu.async_copy(h_hbm.at[nxt_s[b]], rows[b], gsem[b])

    def ring(g, carry):
      @pl.when(g % 2 == 0)
      def _even():
        half(g, set_a, set_b)

      @pl.when(g % 2 == 1)
      def _odd():
        half(g, set_b, set_a)
      return carry
    lax.fori_loop(0, G_ITERS, ring, 0)

    @pl.when(t < G_LEFT)
    def _tail():
      base = (t + G_FULL * NW) * CH
      pltpu.sync_copy(src_hbm.at[pl.ds(base, CH)], sa0)
      pltpu.sync_copy(dst_hbm.at[pl.ds(base, CH)], da0)
      pltpu.async_copy(h_hbm.at[sa0], rows0, gsem0).wait()
      pltpu.sync_copy(rows0, agg_sh.at[da0], add=True)
      if with_deg:
        pltpu.sync_copy(ones1, deg_sh.at[da0], add=True)

    plsc.subcore_barrier()

    n_co = ROWS_PT // CH

    def rslice(i):
      return agg_sh.at[pl.ds(s * ROWS_PT + i * CH, CH)]

    def wslice(i):
      return agg_out.at[pl.ds(c * N_PAD + s * ROWS_PT + i * CH, CH)]

    rcp = [None, None]
    wcp = [None, None]
    rcp[0] = pltpu.async_copy(rslice(0), rows0, gsem0)
    robuf = (rows0, rows1)
    for i in range(n_co):
      b = i % 2
      nb = 1 - b
      rcp[b].wait()
      wcp[b] = pltpu.async_copy(robuf[b], wslice(i), ssem[b])
      if i + 1 < n_co:
        if wcp[nb] is not None:
          wcp[nb].wait()
        rcp[nb] = pltpu.async_copy(rslice(i + 1), robuf[nb], gsem[nb])
    wcp[(n_co - 1) % 2].wait()
    if n_co > 1:
      wcp[(n_co - 2) % 2].wait()

    if with_deg:
      def cd(i, carry):
        r0 = s * ROWS_PT + i * CH
        pltpu.sync_copy(deg_sh.at[pl.ds(r0, CH)], ones1)
        pltpu.sync_copy(ones1, deg_out.at[pl.ds(c * N_PAD + r0, CH)])
        return carry
      lax.fori_loop(0, ROWS_PT // CH, cd, 0)

  return body


def _make_agg(with_deg):
  out_type = [jax.ShapeDtypeStruct((NC * N_PAD, D), jnp.float32)]
  scratch = (
      [pltpu.VMEM((CH,), jnp.int32) for _ in range(8)]
      + [pltpu.VMEM((CH, D), jnp.float32) for _ in range(2)]
  )
  if with_deg:
    out_type.append(jax.ShapeDtypeStruct((NC * N_PAD,), jnp.float32))
    scratch = scratch + [pltpu.VMEM((CH,), jnp.float32)]
  scratch = scratch + [pltpu.VMEM_SHARED((N_PAD, D), jnp.float32)]
  if with_deg:
    scratch = scratch + [pltpu.VMEM_SHARED((N_PAD,), jnp.float32)]
  scratch = scratch + [pltpu.SemaphoreType.DMA
                       for _ in range(8 if with_deg else 6)]
  return pl.kernel(
      _make_agg_body(with_deg),
      out_type=out_type,
      mesh=plsc.VectorSubcoreMesh(core_axis_name="c", subcore_axis_name="s"),
      scratch_types=scratch,
  )


_agg = _make_agg(False)
_agg_deg = _make_agg(True)


_pairs_mesh = plsc.VectorSubcoreMesh(core_axis_name="c", subcore_axis_name="s")


def _pairs_body(h_hbm, a_hbm, b_hbm, a_out, b_out,
                aidx0, aidx1, bidx0, bidx1, ra0, ra1, rb0, rb1,
                gsem0, gsem1, osem0, osem1):
  c = lax.axis_index("c")
  s = lax.axis_index("s")
  t = s * NC + c
  aidx = (aidx0, aidx1)
  bidx = (bidx0, bidx1)
  ra = (ra0, ra1)
  rb = (rb0, rb1)
  gsem = (gsem0, gsem1)
  osem = (osem0, osem1)

  def chunk_pair(g, carry):
    for b in range(2):
      i = g * 2 + b
      base = (t * PNCH + i) * PCH

      @pl.when(g > 0)
      def _drain():
        pltpu.make_async_copy(ra[b], a_out.at[pl.ds(0, PCH)], osem[b]).wait()
        pltpu.make_async_copy(rb[b], b_out.at[pl.ds(0, PCH)], osem[b]).wait()

      pltpu.sync_copy(a_hbm.at[pl.ds(base, PCH)], aidx[b])
      pltpu.sync_copy(b_hbm.at[pl.ds(base, PCH)], bidx[b])
      pltpu.async_copy(h_hbm.at[aidx[b]], ra[b], gsem[b])
      pltpu.async_copy(h_hbm.at[bidx[b]], rb[b], gsem[b])
    for b in range(2):
      i = g * 2 + b
      base = (t * PNCH + i) * PCH
      pltpu.make_async_copy(h_hbm.at[pl.ds(0, PCH)], ra[b], gsem[b]).wait()
      pltpu.make_async_copy(h_hbm.at[pl.ds(0, PCH)], rb[b], gsem[b]).wait()
      pltpu.async_copy(ra[b], a_out.at[pl.ds(base, PCH)], osem[b])
      pltpu.async_copy(rb[b], b_out.at[pl.ds(base, PCH)], osem[b])
    return carry
  lax.fori_loop(0, PNCH // 2, chunk_pair, 0)

  for b in range(2):
    pltpu.make_async_copy(ra[b], a_out.at[pl.ds(0, PCH)], osem[b]).wait()
    pltpu.make_async_copy(rb[b], b_out.at[pl.ds(0, PCH)], osem[b]).wait()


_pairs = pl.kernel(
    _pairs_body,
    out_type=[jax.ShapeDtypeStruct((P_TOT, D), jnp.float32),
              jax.ShapeDtypeStruct((P_TOT, D), jnp.float32)],
    mesh=_pairs_mesh,
    scratch_types=(
        [pltpu.VMEM((PCH,), jnp.int32) for _ in range(4)]
        + [pltpu.VMEM((PCH, D), jnp.float32) for _ in range(4)]
        + [pltpu.SemaphoreType.DMA for _ in range(4)]
    ),
)


def _layer_tc(h, parts, deg2, Ws, Wn, b, relu):
  n = h.shape[0]
  bm = 1000

  def body(h_ref, p_ref, d_ref, ws_ref, wn_ref, b_ref, o_ref):
    dcol = d_ref[0] + d_ref[1]
    hn = (p_ref[0] + p_ref[1]) / jnp.maximum(dcol, 1.0)
    acc = jnp.dot(h_ref[...], ws_ref[...], preferred_element_type=jnp.float32)
    acc = acc + jnp.dot(hn, wn_ref[...], preferred_element_type=jnp.float32)
    acc = acc + b_ref[...]
    if relu:
      acc = jnp.maximum(acc, 0.0)
    o_ref[...] = acc

  return pl.pallas_call(
      body,
      grid=(n // bm,),
      in_specs=[
          pl.BlockSpec((bm, D), lambda i: (i, 0)),
          pl.BlockSpec((NC, bm, D), lambda i: (0, i, 0)),
          pl.BlockSpec((NC, bm, 1), lambda i: (0, i, 0)),
          pl.BlockSpec((D, D), lambda i: (0, 0)),
          pl.BlockSpec((D, D), lambda i: (0, 0)),
          pl.BlockSpec((1, D), lambda i: (0, 0)),
      ],
      out_specs=pl.BlockSpec((bm, D), lambda i: (i, 0)),
      out_shape=jax.ShapeDtypeStruct((n, D), jnp.float32),
  )(h, parts, deg2, Ws, Wn, b)


def _pred_tc(arows, brows, W1, c1, W2, c2, W3p, c3p):
  m = arows.shape[0]
  bm = 4096

  def body(a_ref, b_ref, w1, b1, w2, b2, w3, b3, o_ref):
    x = a_ref[...] * b_ref[...]
    h1 = jnp.dot(x, w1[...], preferred_element_type=jnp.float32)
    h1 = jnp.maximum(h1 + b1[...], 0.0)
    h2 = jnp.dot(h1, w2[...], preferred_element_type=jnp.float32)
    h2 = jnp.maximum(h2 + b2[...], 0.0)
    o_ref[...] = jnp.dot(h2, w3[...], preferred_element_type=jnp.float32) + b3[...]

  return pl.pallas_call(
      body,
      grid=(m // bm,),
      in_specs=[
          pl.BlockSpec((bm, D), lambda i: (i, 0)),
          pl.BlockSpec((bm, D), lambda i: (i, 0)),
          pl.BlockSpec((D, D), lambda i: (0, 0)),
          pl.BlockSpec((1, D), lambda i: (0, 0)),
          pl.BlockSpec((D, D), lambda i: (0, 0)),
          pl.BlockSpec((1, D), lambda i: (0, 0)),
          pl.BlockSpec((D, 8), lambda i: (0, 0)),
          pl.BlockSpec((1, 8), lambda i: (0, 0)),
      ],
      out_specs=pl.BlockSpec((bm, 8), lambda i: (i, 0)),
      out_shape=jax.ShapeDtypeStruct((m, 8), jnp.float32),
  )(arows, brows, W1, c1, W2, c2, W3p, c3p)


def kernel(x, edge_index, pos_edge_index, neg_edge_index,
           W_self1, W_neigh1, b1, W_self2, W_neigh2, b2,
           W_self3, W_neigh3, b3,
           P1_W, P1_b, P2_W, P2_b, P3_W, P3_b):
  src = edge_index[0].astype(jnp.int32)
  dst = edge_index[1].astype(jnp.int32)

  parts1, deg1d = _agg_deg(x, src, dst)
  parts1 = parts1.reshape(NC, N_PAD, D)
  deg2 = deg1d.reshape(NC, N_PAD, 1)
  h1 = _layer_tc(x, parts1, deg2, W_self1, W_neigh1,
                 b1.reshape(1, D), relu=True)
  parts2, = _agg(h1, src, dst)
  parts2 = parts2.reshape(NC, N_PAD, D)
  h2 = _layer_tc(h1, parts2, deg2, W_self2, W_neigh2,
                 b2.reshape(1, D), relu=True)
  parts3, = _agg(h2, src, dst)
  parts3 = parts3.reshape(NC, N_PAD, D)
  h3 = _layer_tc(h2, parts3, deg2, W_self3, W_neigh3,
                 b3.reshape(1, D), relu=False)

  pair = jnp.concatenate([pos_edge_index, neg_edge_index], axis=1)
  a_idx = pair[0].astype(jnp.int32)
  b_idx = pair[1].astype(jnp.int32)
  arows, brows = _pairs(h3, a_idx, b_idx)

  W3p = jnp.pad(P3_W, ((0, 0), (0, 7)))
  c3p = jnp.pad(P3_b, (0, 7)).reshape(1, 8)
  out = _pred_tc(arows, brows, P1_W, P1_b.reshape(1, D),
                 P2_W, P2_b.reshape(1, D), W3p, c3p)
  col = out[:, 0:1]
  return (col[:N_PAIRS], col[N_PAIRS:])
